# baseline (speedup 1.0000x reference)
import functools

import jax
import jax.numpy as jnp
from jax.experimental import pallas as pl
from jax.experimental.pallas import tpu as pltpu

NUM_EXPERTS = 8
TOP_K = 2
HIDDEN = 1024
INTERMEDIATE = 1024
SWIGLU_LIMIT = 7.0
SWIGLU_ALPHA = 1.702
TOKENS = 1024
BT = 256


def _moe_body(rl_ref, x_ref, gup_ref, gub_ref, dp_ref, dpb_ref, out_ref):
    e = pl.program_id(1)
    x = x_ref[...]
    h = jnp.dot(x, gup_ref[0], preferred_element_type=jnp.float32)
    h += gub_ref[0, 0][None, :]
    g = h[:, :INTERMEDIATE]
    l = h[:, INTERMEDIATE:]
    g = jnp.minimum(g, SWIGLU_LIMIT)
    l = jnp.clip(l, -SWIGLU_LIMIT, SWIGLU_LIMIT)
    s = g * jax.nn.sigmoid(SWIGLU_ALPHA * g) * (l + 1.0)
    y = jnp.dot(s.astype(jnp.bfloat16), dp_ref[0],
                preferred_element_type=jnp.float32)
    y += dpb_ref[0, 0][None, :]

    logits = rl_ref[...]
    mx = jnp.max(logits, axis=1, keepdims=True)
    ex = jnp.exp(logits - mx)
    probs = ex / jnp.sum(ex, axis=1, keepdims=True)
    idx = jax.lax.broadcasted_iota(jnp.int32, probs.shape, 1)
    m1 = jnp.max(probs, axis=1, keepdims=True)
    i1 = jnp.min(jnp.where(probs == m1, idx, NUM_EXPERTS), axis=1,
                 keepdims=True)
    p2 = jnp.where(idx == i1, -jnp.inf, probs)
    m2 = jnp.max(p2, axis=1, keepdims=True)
    i2 = jnp.min(jnp.where(p2 == m2, idx, NUM_EXPERTS), axis=1,
                 keepdims=True)
    denom = m1 + m2
    c = jnp.where(i1 == e, m1 / denom, jnp.where(i2 == e, m2 / denom, 0.0))

    @pl.when(e == 0)
    def _():
        out_ref[...] = jnp.zeros_like(out_ref)

    out_ref[...] += c * y


def kernel(hidden_states, router_logits, gate_up_proj, gate_up_proj_bias,
           down_proj, down_proj_bias):
    gup = jnp.concatenate(
        [gate_up_proj[:, :, 0::2], gate_up_proj[:, :, 1::2]], axis=-1
    ).astype(jnp.bfloat16)
    gub = jnp.concatenate(
        [gate_up_proj_bias[:, 0::2], gate_up_proj_bias[:, 1::2]], axis=-1)
    nt = TOKENS // BT
    return pl.pallas_call(
        _moe_body,
        grid=(nt, NUM_EXPERTS),
        in_specs=[
            pl.BlockSpec((BT, NUM_EXPERTS), lambda i, e: (i, 0)),
            pl.BlockSpec((BT, HIDDEN), lambda i, e: (i, 0)),
            pl.BlockSpec((1, HIDDEN, 2 * INTERMEDIATE), lambda i, e: (e, 0, 0)),
            pl.BlockSpec((1, 1, 2 * INTERMEDIATE), lambda i, e: (e, 0, 0)),
            pl.BlockSpec((1, INTERMEDIATE, HIDDEN), lambda i, e: (e, 0, 0)),
            pl.BlockSpec((1, 1, HIDDEN), lambda i, e: (e, 0, 0)),
        ],
        out_specs=pl.BlockSpec((BT, HIDDEN), lambda i, e: (i, 0)),
        out_shape=jax.ShapeDtypeStruct((TOKENS, HIDDEN), jnp.float32),
    )(router_logits, hidden_states.astype(jnp.bfloat16), gup,
      gub[:, None, :], down_proj.astype(jnp.bfloat16),
      down_proj_bias[:, None, :])


# speedup vs baseline: 5.0412x; 5.0412x over previous
import functools

import jax
import jax.numpy as jnp
from jax.experimental import pallas as pl
from jax.experimental.pallas import tpu as pltpu

NUM_EXPERTS = 8
TOP_K = 2
HIDDEN = 1024
INTERMEDIATE = 1024
SWIGLU_LIMIT = 7.0
SWIGLU_ALPHA = 1.702
TOKENS = 1024
BT = 256


def _moe_body(rl_ref, x_ref, gup_ref, gub_ref, dp2_ref, dpb_ref, out_ref):
    e = pl.program_id(1)
    x = x_ref[...]
    h = jnp.dot(x, gup_ref[0], preferred_element_type=jnp.float32)
    h += gub_ref[0, 0][None, :]
    a = jnp.minimum(h, SWIGLU_LIMIT)
    a = a * jax.nn.sigmoid(SWIGLU_ALPHA * a)
    b = jnp.clip(h, -SWIGLU_LIMIT, SWIGLU_LIMIT) + 1.0
    s = a * pltpu.roll(b, 2 * INTERMEDIATE - 1, axis=1)
    lane = jax.lax.broadcasted_iota(jnp.int32, s.shape, 1)
    s = jnp.where(lane % 2 == 0, s, 0.0)
    y = jnp.dot(s.astype(jnp.bfloat16), dp2_ref[0],
                preferred_element_type=jnp.float32)
    y += dpb_ref[0, 0][None, :]

    logits = rl_ref[...]
    mx = jnp.max(logits, axis=1, keepdims=True)
    ex = jnp.exp(logits - mx)
    probs = ex / jnp.sum(ex, axis=1, keepdims=True)
    idx = jax.lax.broadcasted_iota(jnp.int32, probs.shape, 1)
    m1 = jnp.max(probs, axis=1, keepdims=True)
    i1 = jnp.min(jnp.where(probs == m1, idx, NUM_EXPERTS), axis=1,
                 keepdims=True)
    p2 = jnp.where(idx == i1, -jnp.inf, probs)
    m2 = jnp.max(p2, axis=1, keepdims=True)
    i2 = jnp.min(jnp.where(p2 == m2, idx, NUM_EXPERTS), axis=1,
                 keepdims=True)
    denom = m1 + m2
    c = jnp.where(i1 == e, m1 / denom, jnp.where(i2 == e, m2 / denom, 0.0))

    @pl.when(e == 0)
    def _():
        out_ref[...] = jnp.zeros_like(out_ref)

    out_ref[...] += c * y


def kernel(hidden_states, router_logits, gate_up_proj, gate_up_proj_bias,
           down_proj, down_proj_bias):
    gup = gate_up_proj.astype(jnp.bfloat16)
    dp2 = jnp.repeat(down_proj, 2, axis=1).astype(jnp.bfloat16)
    nt = TOKENS // BT
    return pl.pallas_call(
        _moe_body,
        grid=(nt, NUM_EXPERTS),
        in_specs=[
            pl.BlockSpec((BT, NUM_EXPERTS), lambda i, e: (i, 0)),
            pl.BlockSpec((BT, HIDDEN), lambda i, e: (i, 0)),
            pl.BlockSpec((1, HIDDEN, 2 * INTERMEDIATE), lambda i, e: (e, 0, 0)),
            pl.BlockSpec((1, 1, 2 * INTERMEDIATE), lambda i, e: (e, 0, 0)),
            pl.BlockSpec((1, 2 * INTERMEDIATE, HIDDEN), lambda i, e: (e, 0, 0)),
            pl.BlockSpec((1, 1, HIDDEN), lambda i, e: (e, 0, 0)),
        ],
        out_specs=pl.BlockSpec((BT, HIDDEN), lambda i, e: (i, 0)),
        out_shape=jax.ShapeDtypeStruct((TOKENS, HIDDEN), jnp.float32),
    )(router_logits, hidden_states.astype(jnp.bfloat16), gup,
      gate_up_proj_bias[:, None, :], dp2, down_proj_bias[:, None, :])
